# baseline (device time: 717376 ns/iter reference)
import jax
import jax.numpy as jnp
from jax import lax
from jax.experimental import pallas as pl
from jax.experimental.pallas import tpu as pltpu

N_DEV = 4
N_PIECES = 16


def kernel(x, w_mat):
    m_per, k = x.shape
    n_per = w_mat.shape[1]
    m_total = N_DEV * m_per
    half = m_per // 2
    quar = m_per // 4

    def body(rows_ref, x_ref, w_ref, out_ref, xg, in_buf, out_buf,
             copy_sem, load_sem, store_sem, send_sems, rs1, rs2):
        my = lax.axis_index("i")
        left = (my - 1) % N_DEV
        right = (my + 1) % N_DEV
        diag = (my + 2) % N_DEV

        barrier_sem = pltpu.get_barrier_semaphore()
        for nbr in (left, right):
            pl.semaphore_signal(
                barrier_sem, inc=1, device_id=(nbr,),
                device_id_type=pl.DeviceIdType.MESH,
            )
        pl.semaphore_wait(barrier_sem, 2)

        def send(src_ref, src_row, dst_row, nrows, dev, si, rsem):
            rd = pltpu.make_async_remote_copy(
                src_ref=src_ref.at[pl.ds(src_row, nrows), :],
                dst_ref=xg.at[pl.ds(dst_row, nrows), :],
                send_sem=send_sems.at[si],
                recv_sem=rsem,
                device_id=(dev,),
                device_id_type=pl.DeviceIdType.MESH,
            )
            rd.start()

        def recv_wait(row_start, nrows, rsem):
            pltpu.make_async_remote_copy(
                src_ref=x_ref.at[pl.ds(0, nrows), :],
                dst_ref=xg.at[pl.ds(row_start, nrows), :],
                send_sem=send_sems.at[0],
                recv_sem=rsem,
                device_id=(left,),
                device_id_type=pl.DeviceIdType.MESH,
            ).wait_recv()

        si = 0
        for h in range(2):
            for dir_idx, dev in ((0, right), (1, left)):
                send(x_ref, h * half, my * m_per + h * half, half,
                     dev, si, rs1.at[dir_idx, h])
                si += 1

        cp = pltpu.make_async_copy(
            x_ref, xg.at[pl.ds(my * m_per, m_per), :], copy_sem
        )
        cp.start()
        cp.wait()

        def step(i, carry):
            @pl.when(i == 4)
            def _():
                recv_wait(left * m_per, half, rs1.at[0, 0])
                for q in range(2):
                    send(xg, left * m_per + q * quar,
                         left * m_per + q * quar,
                         quar, right, 4 + q, rs2.at[0, q])

            @pl.when(i == 6)
            def _():
                recv_wait(right * m_per, half, rs1.at[1, 0])

            @pl.when(i == 8)
            def _():
                recv_wait(right * m_per + half, half, rs1.at[1, 1])
                for q in range(2):
                    send(xg, right * m_per + half + q * quar,
                         right * m_per + half + q * quar,
                         quar, left, 6 + q, rs2.at[1, q])

            @pl.when(i == 10)
            def _():
                recv_wait(left * m_per + half, half, rs1.at[0, 1])

            @pl.when(i == 12)
            def _():
                recv_wait(diag * m_per, quar, rs2.at[0, 0])

            @pl.when(i == 13)
            def _():
                recv_wait(diag * m_per + half, quar, rs2.at[1, 0])

            @pl.when(i == 14)
            def _():
                recv_wait(diag * m_per + quar, quar, rs2.at[0, 1])

            @pl.when(i == 15)
            def _():
                recv_wait(diag * m_per + half + quar, quar,
                          rs2.at[1, 1])

            row = pl.multiple_of(rows_ref[i], quar)
            ld = pltpu.make_async_copy(
                xg.at[pl.ds(row, quar), :], in_buf, load_sem
            )
            ld.start()
            ld.wait()
            y = jnp.dot(in_buf[...], w_ref[...],
                        preferred_element_type=jnp.float32)
            out_buf[...] = y * (1.0 / (1.0 + jnp.exp(-y)))
            st = pltpu.make_async_copy(
                out_buf, out_ref.at[pl.ds(row, quar), :], store_sem
            )
            st.start()
            st.wait()
            return carry

        lax.fori_loop(0, N_PIECES, step, 0)

        for j in range(4):
            pltpu.make_async_remote_copy(
                src_ref=x_ref.at[pl.ds(0, half), :],
                dst_ref=xg.at[pl.ds(0, half), :],
                send_sem=send_sems.at[j],
                recv_sem=rs1.at[0, 0],
                device_id=(left,),
                device_id_type=pl.DeviceIdType.MESH,
            ).wait_send()
        for j in range(4, 8):
            pltpu.make_async_remote_copy(
                src_ref=x_ref.at[pl.ds(0, quar), :],
                dst_ref=xg.at[pl.ds(0, quar), :],
                send_sem=send_sems.at[j],
                recv_sem=rs1.at[0, 0],
                device_id=(left,),
                device_id_type=pl.DeviceIdType.MESH,
            ).wait_send()

    my = lax.axis_index("i")
    left = (my - 1) % N_DEV
    right = (my + 1) % N_DEV
    diag = (my + 2) % N_DEV
    order = [
        (my, 0), (my, 1), (my, 2), (my, 3),
        (left, 0), (left, 1), (right, 0), (right, 1),
        (right, 2), (right, 3), (left, 2), (left, 3),
        (diag, 0), (diag, 2), (diag, 1), (diag, 3),
    ]
    rows = jnp.stack([c * m_per + q * quar for c, q in order]).astype(jnp.int32)

    out, _xg = pl.pallas_call(
        body,
        out_shape=[
            jax.ShapeDtypeStruct((m_total, n_per), jnp.float32),
            jax.ShapeDtypeStruct((m_total, k), jnp.float32),
        ],
        in_specs=[
            pl.BlockSpec(memory_space=pltpu.SMEM),
            pl.BlockSpec(memory_space=pl.ANY),
            pl.BlockSpec(memory_space=pltpu.VMEM),
        ],
        out_specs=[
            pl.BlockSpec(memory_space=pl.ANY),
            pl.BlockSpec(memory_space=pl.ANY),
        ],
        scratch_shapes=[
            pltpu.VMEM((quar, k), jnp.float32),
            pltpu.VMEM((quar, n_per), jnp.float32),
            pltpu.SemaphoreType.DMA,
            pltpu.SemaphoreType.DMA,
            pltpu.SemaphoreType.DMA,
            pltpu.SemaphoreType.DMA((8,)),
            pltpu.SemaphoreType.DMA((2, 2)),
            pltpu.SemaphoreType.DMA((2, 2)),
        ],
        compiler_params=pltpu.CompilerParams(
            collective_id=0,
            vmem_limit_bytes=60 * 1024 * 1024,
        ),
    )(rows, x, w_mat)
    return out


# device time: 696567 ns/iter; 1.0299x vs baseline; 1.0299x over previous
import jax
import jax.numpy as jnp
from jax import lax
from jax.experimental import pallas as pl
from jax.experimental.pallas import tpu as pltpu

N_DEV = 4


def kernel(x, w_mat):
    m_per, k = x.shape
    n_per = w_mat.shape[1]
    m_total = N_DEV * m_per
    half = m_per // 2
    quar = m_per // 4

    def body(rows_ref, x_ref, w_ref, out_ref, xg, in_buf, out_buf,
             copy_sem, load_sem, store_sem, send_sems, rs1, rs2):
        my = lax.axis_index("i")
        left = (my - 1) % N_DEV
        right = (my + 1) % N_DEV
        diag = (my + 2) % N_DEV

        barrier_sem = pltpu.get_barrier_semaphore()
        for nbr in (left, right):
            pl.semaphore_signal(
                barrier_sem, inc=1, device_id=(nbr,),
                device_id_type=pl.DeviceIdType.MESH,
            )
        pl.semaphore_wait(barrier_sem, 2)

        def send(src_ref, src_row, dst_row, nrows, dev, si, rsem):
            rd = pltpu.make_async_remote_copy(
                src_ref=src_ref.at[pl.ds(src_row, nrows), :],
                dst_ref=xg.at[pl.ds(dst_row, nrows), :],
                send_sem=send_sems.at[si],
                recv_sem=rsem,
                device_id=(dev,),
                device_id_type=pl.DeviceIdType.MESH,
            )
            rd.start()

        def recv_wait(row_start, nrows, rsem):
            pltpu.make_async_remote_copy(
                src_ref=x_ref.at[pl.ds(0, nrows), :],
                dst_ref=xg.at[pl.ds(row_start, nrows), :],
                send_sem=send_sems.at[0],
                recv_sem=rsem,
                device_id=(left,),
                device_id_type=pl.DeviceIdType.MESH,
            ).wait_recv()

        si = 0
        for h in range(2):
            for dir_idx, dev in ((0, right), (1, left)):
                send(x_ref, h * half, my * m_per + h * half, half,
                     dev, si, rs1.at[dir_idx, h])
                si += 1

        cp = pltpu.make_async_copy(
            x_ref, xg.at[pl.ds(my * m_per, m_per), :], copy_sem
        )
        cp.start()
        cp.wait()

        def step(i, carry):
            @pl.when(i == 1)
            def _():
                recv_wait(left * m_per, half, rs1.at[0, 0])
                for q in range(2):
                    send(xg, left * m_per + q * quar,
                         left * m_per + q * quar,
                         quar, right, 4 + q, rs2.at[0, q])
                recv_wait(right * m_per + half, half, rs1.at[1, 1])
                for q in range(2):
                    send(xg, right * m_per + half + q * quar,
                         right * m_per + half + q * quar,
                         quar, left, 6 + q, rs2.at[1, q])
                recv_wait(left * m_per + half, half, rs1.at[0, 1])

            @pl.when(i == 2)
            def _():
                recv_wait(right * m_per, half, rs1.at[1, 0])

            @pl.when(i == 3)
            def _():
                for dir_idx, q in ((0, 0), (1, 2), (0, 1), (1, 3)):
                    recv_wait(diag * m_per + q * quar, quar,
                              rs2.at[dir_idx, q % 2])

            row = pl.multiple_of(rows_ref[i], quar)
            ld = pltpu.make_async_copy(
                xg.at[pl.ds(row, m_per), :], in_buf, load_sem
            )
            ld.start()
            ld.wait()
            y = jnp.dot(in_buf[...], w_ref[...],
                        preferred_element_type=jnp.float32)
            out_buf[...] = y * (1.0 / (1.0 + jnp.exp(-y)))
            st = pltpu.make_async_copy(
                out_buf, out_ref.at[pl.ds(row, m_per), :], store_sem
            )
            st.start()
            st.wait()
            return carry

        lax.fori_loop(0, N_DEV, step, 0)

        for j in range(4):
            pltpu.make_async_remote_copy(
                src_ref=x_ref.at[pl.ds(0, half), :],
                dst_ref=xg.at[pl.ds(0, half), :],
                send_sem=send_sems.at[j],
                recv_sem=rs1.at[0, 0],
                device_id=(left,),
                device_id_type=pl.DeviceIdType.MESH,
            ).wait_send()
        for j in range(4, 8):
            pltpu.make_async_remote_copy(
                src_ref=x_ref.at[pl.ds(0, quar), :],
                dst_ref=xg.at[pl.ds(0, quar), :],
                send_sem=send_sems.at[j],
                recv_sem=rs1.at[0, 0],
                device_id=(left,),
                device_id_type=pl.DeviceIdType.MESH,
            ).wait_send()

    my = lax.axis_index("i")
    order = [my, (my - 1) % N_DEV, (my + 1) % N_DEV, (my + 2) % N_DEV]
    rows = jnp.stack([c * m_per for c in order]).astype(jnp.int32)

    out, _xg = pl.pallas_call(
        body,
        out_shape=[
            jax.ShapeDtypeStruct((m_total, n_per), jnp.float32),
            jax.ShapeDtypeStruct((m_total, k), jnp.float32),
        ],
        in_specs=[
            pl.BlockSpec(memory_space=pltpu.SMEM),
            pl.BlockSpec(memory_space=pl.ANY),
            pl.BlockSpec(memory_space=pltpu.VMEM),
        ],
        out_specs=[
            pl.BlockSpec(memory_space=pl.ANY),
            pl.BlockSpec(memory_space=pl.ANY),
        ],
        scratch_shapes=[
            pltpu.VMEM((m_per, k), jnp.float32),
            pltpu.VMEM((m_per, n_per), jnp.float32),
            pltpu.SemaphoreType.DMA,
            pltpu.SemaphoreType.DMA,
            pltpu.SemaphoreType.DMA,
            pltpu.SemaphoreType.DMA((8,)),
            pltpu.SemaphoreType.DMA((2, 2)),
            pltpu.SemaphoreType.DMA((2, 2)),
        ],
        compiler_params=pltpu.CompilerParams(
            collective_id=0,
            vmem_limit_bytes=60 * 1024 * 1024,
        ),
    )(rows, x, w_mat)
    return out


# device time: 340242 ns/iter; 2.1084x vs baseline; 2.0473x over previous
import jax
import jax.numpy as jnp
from jax import lax
from jax.experimental import pallas as pl
from jax.experimental.pallas import tpu as pltpu

jax.config.update("jax_compilation_cache_dir", "/tmp/scband_jax_cache")
jax.config.update("jax_persistent_cache_min_compile_time_secs", 0.0)
jax.config.update("jax_persistent_cache_min_entry_size_bytes", 0)

N_DEV = 4


def kernel(x, w_mat):
    m_per, k = x.shape
    n_per = w_mat.shape[1]
    m_total = N_DEV * m_per
    half = m_per // 2
    quar = m_per // 4

    def body(x_ref, w_ref, out_ref, xg, in_buf, out_buf,
             load_sems, store_sems, send_sems, rs1, rs2):
        my = lax.axis_index("i")
        left = (my - 1) % N_DEV
        right = (my + 1) % N_DEV
        diag = (my + 2) % N_DEV

        barrier_sem = pltpu.get_barrier_semaphore()
        for nbr in (left, right):
            pl.semaphore_signal(
                barrier_sem, inc=1, device_id=(nbr,),
                device_id_type=pl.DeviceIdType.MESH,
            )
        pl.semaphore_wait(barrier_sem, 2)

        sends = []

        def send(src_ref, src_row, dst_row, nrows, dev, si, rsem):
            rd = pltpu.make_async_remote_copy(
                src_ref=src_ref.at[pl.ds(src_row, nrows), :],
                dst_ref=xg.at[pl.ds(dst_row, nrows), :],
                send_sem=send_sems.at[si],
                recv_sem=rsem,
                device_id=(dev,),
                device_id_type=pl.DeviceIdType.MESH,
            )
            rd.start()
            sends.append(rd)

        def recv_wait(row_start, nrows, rsem):
            pltpu.make_async_remote_copy(
                src_ref=x_ref.at[pl.ds(0, nrows), :],
                dst_ref=xg.at[pl.ds(row_start, nrows), :],
                send_sem=send_sems.at[0],
                recv_sem=rsem,
                device_id=(left,),
                device_id_type=pl.DeviceIdType.MESH,
            ).wait_recv()

        si = 0
        for h in range(2):
            for dir_idx, dev in ((0, right), (1, left)):
                send(x_ref, h * half, my * m_per + h * half, half,
                     dev, si, rs1.at[dir_idx, h])
                si += 1

        state = {"pending": [None]}

        def compute_chunk(src_ref, src_row, out_row):
            ld = pltpu.make_async_copy(
                src_ref.at[pl.ds(src_row, m_per), :],
                in_buf, load_sems.at[0],
            )
            ld.start()
            ld.wait()
            if state["pending"][0] is not None:
                state["pending"][0].wait()
            y = jnp.dot(in_buf[...], w_ref[...],
                        preferred_element_type=jnp.float32)
            out_buf[...] = y * (1.0 / (1.0 + jnp.exp(-y)))
            st = pltpu.make_async_copy(
                out_buf,
                out_ref.at[pl.ds(out_row, m_per), :],
                store_sems.at[0],
            )
            st.start()
            state["pending"][0] = st

        compute_chunk(x_ref, 0, my * m_per)

        recv_wait(left * m_per, half, rs1.at[0, 0])
        for q in range(2):
            send(xg, left * m_per + q * quar, left * m_per + q * quar,
                 quar, right, 4 + q, rs2.at[0, q])

        recv_wait(right * m_per + half, half, rs1.at[1, 1])
        for q in range(2):
            send(xg, right * m_per + half + q * quar,
                 right * m_per + half + q * quar,
                 quar, left, 6 + q, rs2.at[1, q])

        recv_wait(left * m_per + half, half, rs1.at[0, 1])
        compute_chunk(xg, left * m_per, left * m_per)
        recv_wait(right * m_per, half, rs1.at[1, 0])
        compute_chunk(xg, right * m_per, right * m_per)

        for dir_idx, q in ((0, 0), (1, 2), (0, 1), (1, 3)):
            recv_wait(diag * m_per + q * quar, quar,
                      rs2.at[dir_idx, q % 2])
        compute_chunk(xg, diag * m_per, diag * m_per)

        for rd in sends:
            rd.wait_send()
        for p in state["pending"]:
            if p is not None:
                p.wait()

    out, _xg = pl.pallas_call(
        body,
        out_shape=[
            jax.ShapeDtypeStruct((m_total, n_per), jnp.float32),
            jax.ShapeDtypeStruct((m_total, k), jnp.float32),
        ],
        in_specs=[
            pl.BlockSpec(memory_space=pl.ANY),
            pl.BlockSpec(memory_space=pltpu.VMEM),
        ],
        out_specs=[
            pl.BlockSpec(memory_space=pl.ANY),
            pl.BlockSpec(memory_space=pl.ANY),
        ],
        scratch_shapes=[
            pltpu.VMEM((m_per, k), jnp.float32),
            pltpu.VMEM((m_per, n_per), jnp.float32),
            pltpu.SemaphoreType.DMA((1,)),
            pltpu.SemaphoreType.DMA((1,)),
            pltpu.SemaphoreType.DMA((8,)),
            pltpu.SemaphoreType.DMA((2, 2)),
            pltpu.SemaphoreType.DMA((2, 2)),
        ],
        compiler_params=pltpu.CompilerParams(
            collective_id=0,
            vmem_limit_bytes=60 * 1024 * 1024,
        ),
    )(x, w_mat)
    return out


# device time: 326194 ns/iter; 2.1992x vs baseline; 1.0431x over previous
import jax
import jax.numpy as jnp
from jax import lax
from jax.experimental import pallas as pl
from jax.experimental.pallas import tpu as pltpu

jax.config.update("jax_compilation_cache_dir", "/tmp/scband_jax_cache")
jax.config.update("jax_persistent_cache_min_compile_time_secs", 0.0)
jax.config.update("jax_persistent_cache_min_entry_size_bytes", 0)

N_DEV = 4


def kernel(x, w_mat):
    m_per, k = x.shape
    n_per = w_mat.shape[1]
    m_total = N_DEV * m_per
    half = m_per // 2
    quar = m_per // 4

    def body(x_ref, w_ref, out_ref, xg, in_buf, out_buf,
             load_sems, store_sems, send_sems, rs1, rs2):
        my = lax.axis_index("i")
        left = (my - 1) % N_DEV
        right = (my + 1) % N_DEV
        diag = (my + 2) % N_DEV

        barrier_sem = pltpu.get_barrier_semaphore()
        for nbr in (left, right):
            pl.semaphore_signal(
                barrier_sem, inc=1, device_id=(nbr,),
                device_id_type=pl.DeviceIdType.MESH,
            )
        pl.semaphore_wait(barrier_sem, 2)

        sends = []

        def send(src_ref, src_row, dst_row, nrows, dev, si, rsem):
            rd = pltpu.make_async_remote_copy(
                src_ref=src_ref.at[pl.ds(src_row, nrows), :],
                dst_ref=xg.at[pl.ds(dst_row, nrows), :],
                send_sem=send_sems.at[si],
                recv_sem=rsem,
                device_id=(dev,),
                device_id_type=pl.DeviceIdType.MESH,
            )
            rd.start()
            sends.append(rd)

        def recv_wait(row_start, nrows, rsem):
            pltpu.make_async_remote_copy(
                src_ref=x_ref.at[pl.ds(0, nrows), :],
                dst_ref=xg.at[pl.ds(row_start, nrows), :],
                send_sem=send_sems.at[0],
                recv_sem=rsem,
                device_id=(left,),
                device_id_type=pl.DeviceIdType.MESH,
            ).wait_recv()

        si = 0
        for h in range(2):
            for dir_idx, dev in ((0, right), (1, left)):
                send(x_ref, h * half, my * m_per + h * half, half,
                     dev, si, rs1.at[dir_idx, h])
                si += 1

        state = {"slot": 0, "pending": [None, None]}

        def compute_q(src_ref, src_row, out_row):
            slot = state["slot"]
            state["slot"] = 1 - slot
            ld = pltpu.make_async_copy(
                src_ref.at[pl.ds(src_row, quar), :],
                in_buf.at[slot], load_sems.at[slot],
            )
            ld.start()
            ld.wait()
            if state["pending"][slot] is not None:
                state["pending"][slot].wait()
            y = jnp.dot(in_buf[slot], w_ref[...],
                        preferred_element_type=jnp.float32)
            out_buf[slot] = y * (1.0 / (1.0 + jnp.exp(-y)))
            st = pltpu.make_async_copy(
                out_buf.at[slot],
                out_ref.at[pl.ds(out_row, quar), :],
                store_sems.at[slot],
            )
            st.start()
            state["pending"][slot] = st

        for q in range(4):
            compute_q(x_ref, q * quar, my * m_per + q * quar)

        recv_wait(left * m_per, half, rs1.at[0, 0])
        for q in range(2):
            send(xg, left * m_per + q * quar, left * m_per + q * quar,
                 quar, right, 4 + q, rs2.at[0, q])
        for q in range(2):
            compute_q(xg, left * m_per + q * quar, left * m_per + q * quar)

        recv_wait(right * m_per, half, rs1.at[1, 0])
        for q in range(2):
            compute_q(xg, right * m_per + q * quar, right * m_per + q * quar)

        recv_wait(right * m_per + half, half, rs1.at[1, 1])
        for q in range(2):
            send(xg, right * m_per + half + q * quar,
                 right * m_per + half + q * quar,
                 quar, left, 6 + q, rs2.at[1, q])

        recv_wait(left * m_per + half, half, rs1.at[0, 1])
        for q in range(2):
            compute_q(xg, left * m_per + half + q * quar,
                      left * m_per + half + q * quar)
        for q in range(2):
            compute_q(xg, right * m_per + half + q * quar,
                      right * m_per + half + q * quar)

        for dir_idx, q in ((0, 0), (1, 2), (0, 1), (1, 3)):
            recv_wait(diag * m_per + q * quar, quar,
                      rs2.at[dir_idx, q % 2])
            compute_q(xg, diag * m_per + q * quar, diag * m_per + q * quar)

        for rd in sends:
            rd.wait_send()
        for p in state["pending"]:
            if p is not None:
                p.wait()

    out, _xg = pl.pallas_call(
        body,
        out_shape=[
            jax.ShapeDtypeStruct((m_total, n_per), jnp.float32),
            jax.ShapeDtypeStruct((m_total, k), jnp.float32),
        ],
        in_specs=[
            pl.BlockSpec(memory_space=pl.ANY),
            pl.BlockSpec(memory_space=pltpu.VMEM),
        ],
        out_specs=[
            pl.BlockSpec(memory_space=pl.ANY),
            pl.BlockSpec(memory_space=pl.ANY),
        ],
        scratch_shapes=[
            pltpu.VMEM((2, quar, k), jnp.float32),
            pltpu.VMEM((2, quar, n_per), jnp.float32),
            pltpu.SemaphoreType.DMA((2,)),
            pltpu.SemaphoreType.DMA((2,)),
            pltpu.SemaphoreType.DMA((8,)),
            pltpu.SemaphoreType.DMA((2, 2)),
            pltpu.SemaphoreType.DMA((2, 2)),
        ],
        compiler_params=pltpu.CompilerParams(
            collective_id=0,
            vmem_limit_bytes=60 * 1024 * 1024,
        ),
    )(x, w_mat)
    return out
